# Initial kernel scaffold; baseline (speedup 1.0000x reference)
import jax
import jax.numpy as jnp
from jax.experimental import pallas as pl


def kernel(inputs, codebook):
    raise NotImplementedError("write your pallas kernel here")


# speedup vs baseline: 1.0361x; 1.0361x over previous
import functools

import jax
import jax.numpy as jnp
from jax import lax
from jax.experimental import pallas as pl
from jax.experimental.pallas import tpu as pltpu
from jax.experimental.pallas import tpu_sc as plsc

_K = 8192
_D = 256
_N = 4096
_ROW_TILE = 512
_CHUNK = 1024

_SC_CORES = 2
_SC_SUBCORES = 16


def _distance_argmin_body(x_ref, cb_ref, idx_ref, dmin_ref):
    x = x_ref[...]
    sx = jnp.sum(x * x, axis=1, keepdims=True)
    best_val = jnp.full((_ROW_TILE,), jnp.inf, jnp.float32)
    best_idx = jnp.zeros((_ROW_TILE,), jnp.int32)
    for c in range(_K // _CHUNK):
        e = cb_ref[c * _CHUNK:(c + 1) * _CHUNK, :]
        se = jnp.sum(e * e, axis=1)
        m = lax.dot_general(x, e, (((1,), (1,)), ((), ())),
                            preferred_element_type=jnp.float32)
        d = (sx + se[None, :]) - 2.0 * m
        cmin = jnp.min(d, axis=1)
        cols = lax.broadcasted_iota(jnp.int32, (_ROW_TILE, _CHUNK), 1)
        cidx = jnp.min(jnp.where(d == cmin[:, None], cols, jnp.int32(2**30)),
                       axis=1) + c * _CHUNK
        upd = cmin < best_val
        best_idx = jnp.where(upd, cidx, best_idx)
        best_val = jnp.where(upd, cmin, best_val)
    idx_ref[0, 0, :] = best_idx
    dmin_ref[0, 0, :] = best_val


def _tc_distance_argmin(x_flat, codebook):
    nt = _N // _ROW_TILE
    idx3, dmin3 = pl.pallas_call(
        _distance_argmin_body,
        grid=(nt,),
        in_specs=[
            pl.BlockSpec((_ROW_TILE, _D), lambda t: (t, 0)),
            pl.BlockSpec((_K, _D), lambda t: (0, 0)),
        ],
        out_specs=(
            pl.BlockSpec((1, 1, _ROW_TILE), lambda t: (t, 0, 0)),
            pl.BlockSpec((1, 1, _ROW_TILE), lambda t: (t, 0, 0)),
        ),
        out_shape=(
            jax.ShapeDtypeStruct((nt, 1, _ROW_TILE), jnp.int32),
            jax.ShapeDtypeStruct((nt, 1, _ROW_TILE), jnp.float32),
        ),
    )(x_flat, codebook)
    return idx3.reshape(_N), dmin3.reshape(_N)


def _sc_gather(codebook, indices):
    nw = _SC_CORES * _SC_SUBCORES
    bpw = _N // nw
    mesh = plsc.VectorSubcoreMesh(core_axis_name="c", subcore_axis_name="s")

    @functools.partial(
        pl.kernel, mesh=mesh,
        out_type=jax.ShapeDtypeStruct((_N, _D), jnp.float32),
        scratch_types=[
            pltpu.VMEM((bpw,), jnp.int32),
            pltpu.VMEM((bpw, _D), jnp.float32),
            pltpu.SemaphoreType.DMA,
        ],
    )
    def gather_kernel(table_hbm, idx_hbm, out_hbm, idx_v, rows_v, sem):
        wid = lax.axis_index("s") * _SC_CORES + lax.axis_index("c")
        base = wid * bpw
        pltpu.sync_copy(idx_hbm.at[pl.ds(base, bpw)], idx_v)
        pltpu.async_copy(table_hbm.at[idx_v], rows_v, sem).wait()
        pltpu.sync_copy(rows_v, out_hbm.at[pl.ds(base, bpw)])

    return gather_kernel(codebook, indices)


def kernel(inputs, codebook):
    x_flat = jnp.transpose(inputs, (0, 2, 3, 1)).reshape(_N, _D)
    indices, dmin = _tc_distance_argmin(x_flat, codebook)
    q_flat = _sc_gather(codebook, indices)
    quantized = jnp.transpose(q_flat.reshape(4, 32, 32, _D), (0, 3, 1, 2))
    vq_loss = 1.25 * (jnp.sum(dmin) / jnp.float32(_N * _D))
    return quantized, vq_loss, indices


# speedup vs baseline: 1.1629x; 1.1224x over previous
import functools

import jax
import jax.numpy as jnp
from jax import lax
from jax.experimental import pallas as pl
from jax.experimental.pallas import tpu as pltpu
from jax.experimental.pallas import tpu_sc as plsc

_K = 8192
_D = 256
_N = 4096
_ROW_TILE = 512
_CHUNK = 1024

_SC_CORES = 2
_SC_SUBCORES = 16


def _codebook_norms_body(cb_ref, se_ref):
    e = cb_ref[...]
    se_ref[...] = jnp.sum(e * e, axis=1)


def _codebook_norms(codebook):
    return pl.pallas_call(
        _codebook_norms_body,
        out_shape=jax.ShapeDtypeStruct((_K,), jnp.float32),
    )(codebook)


def _distance_argmin_body(x_ref, cb_ref, se_ref, idx_ref, dmin_ref):
    x2 = 2.0 * x_ref[...]
    sx = jnp.sum(x_ref[...] * x_ref[...], axis=1, keepdims=True)
    best_val = jnp.full((_ROW_TILE,), jnp.inf, jnp.float32)
    best_idx = jnp.zeros((_ROW_TILE,), jnp.int32)
    for c in range(_K // _CHUNK):
        e = cb_ref[c * _CHUNK:(c + 1) * _CHUNK, :]
        se = se_ref[c * _CHUNK:(c + 1) * _CHUNK]
        m2 = lax.dot_general(x2, e, (((1,), (1,)), ((), ())),
                             preferred_element_type=jnp.float32)
        d = (sx + se[None, :]) - m2
        cmin = jnp.min(d, axis=1)
        cols = lax.broadcasted_iota(jnp.int32, (_ROW_TILE, _CHUNK), 1)
        cidx = jnp.min(jnp.where(d == cmin[:, None], cols, jnp.int32(2**30)),
                       axis=1) + c * _CHUNK
        upd = cmin < best_val
        best_idx = jnp.where(upd, cidx, best_idx)
        best_val = jnp.where(upd, cmin, best_val)
    idx_ref[0, 0, :] = best_idx
    dmin_ref[0, 0, :] = best_val


def _tc_distance_argmin(x_flat, codebook, se):
    nt = _N // _ROW_TILE
    idx3, dmin3 = pl.pallas_call(
        _distance_argmin_body,
        grid=(nt,),
        in_specs=[
            pl.BlockSpec((_ROW_TILE, _D), lambda t: (t, 0)),
            pl.BlockSpec((_K, _D), lambda t: (0, 0)),
            pl.BlockSpec((_K,), lambda t: (0,)),
        ],
        out_specs=(
            pl.BlockSpec((1, 1, _ROW_TILE), lambda t: (t, 0, 0)),
            pl.BlockSpec((1, 1, _ROW_TILE), lambda t: (t, 0, 0)),
        ),
        out_shape=(
            jax.ShapeDtypeStruct((nt, 1, _ROW_TILE), jnp.int32),
            jax.ShapeDtypeStruct((nt, 1, _ROW_TILE), jnp.float32),
        ),
        compiler_params=pltpu.CompilerParams(
            dimension_semantics=("parallel",)),
    )(x_flat, codebook, se)
    return idx3.reshape(_N), dmin3.reshape(_N)


def _sc_gather(codebook, indices):
    nw = _SC_CORES * _SC_SUBCORES
    bpw = _N // nw
    mesh = plsc.VectorSubcoreMesh(core_axis_name="c", subcore_axis_name="s")

    @functools.partial(
        pl.kernel, mesh=mesh,
        out_type=jax.ShapeDtypeStruct((_N, _D), jnp.float32),
        scratch_types=[
            pltpu.VMEM((bpw,), jnp.int32),
            pltpu.VMEM((bpw, _D), jnp.float32),
            pltpu.SemaphoreType.DMA,
        ],
    )
    def gather_kernel(table_hbm, idx_hbm, out_hbm, idx_v, rows_v, sem):
        wid = lax.axis_index("s") * _SC_CORES + lax.axis_index("c")
        base = wid * bpw
        pltpu.sync_copy(idx_hbm.at[pl.ds(base, bpw)], idx_v)
        pltpu.async_copy(table_hbm.at[idx_v], rows_v, sem).wait()
        pltpu.sync_copy(rows_v, out_hbm.at[pl.ds(base, bpw)])

    return gather_kernel(codebook, indices)


def kernel(inputs, codebook):
    x_flat = jnp.transpose(inputs, (0, 2, 3, 1)).reshape(_N, _D)
    se = _codebook_norms(codebook)
    indices, dmin = _tc_distance_argmin(x_flat, codebook, se)
    q_flat = _sc_gather(codebook, indices)
    quantized = jnp.transpose(q_flat.reshape(4, 32, 32, _D), (0, 3, 1, 2))
    vq_loss = 1.25 * (jnp.sum(dmin) / jnp.float32(_N * _D))
    return quantized, vq_loss, indices
